# baseline (speedup 1.0000x reference)
import jax
import jax.numpy as jnp
from jax.experimental import pallas as pl
from jax.experimental.pallas import tpu as pltpu

_TILE_B = 1024


def _fused_two_out_kernel(x_ref, u_ref, wx_ref, wu_ref, b_ref, xh_ref, xn_ref):
    acc = jnp.dot(x_ref[...], wx_ref[...], preferred_element_type=jnp.float32)
    acc_u = jnp.dot(u_ref[...], wu_ref[...], preferred_element_type=jnp.float32)
    en0 = xh_ref.shape[1]
    xh_ref[...] = acc[:, :en0] + b_ref[:, :en0]
    xn_ref[...] = acc[:, en0:] + acc_u + b_ref[:, en0:]


def kernel(x, u_stack, w_full):
    T, batch, b0 = u_stack.shape
    en0 = x.shape[1]
    ku = T * b0
    n_out = 2 * en0

    u_flat = jnp.transpose(u_stack, (1, 0, 2)).reshape(batch, ku)
    u_flat = u_flat.astype(jnp.bfloat16)

    wx = w_full[:en0, :n_out]
    wu = w_full[en0:en0 + ku, en0:n_out].astype(jnp.bfloat16)
    bias = w_full[en0 + ku:en0 + ku + 1, :n_out]

    tb = _TILE_B if batch % _TILE_B == 0 else batch
    grid = (batch // tb,)

    cost = pl.CostEstimate(
        flops=2 * batch * (en0 * n_out + ku * en0),
        transcendentals=0,
        bytes_accessed=4 * (batch * (en0 + ku + n_out)
                            + en0 * n_out + ku * en0),
    )

    xh, xn = pl.pallas_call(
        _fused_two_out_kernel,
        out_shape=(
            jax.ShapeDtypeStruct((batch, en0), x.dtype),
            jax.ShapeDtypeStruct((batch, en0), x.dtype),
        ),
        grid=grid,
        in_specs=[
            pl.BlockSpec((tb, en0), lambda i: (i, 0)),
            pl.BlockSpec((tb, ku), lambda i: (i, 0)),
            pl.BlockSpec((en0, n_out), lambda i: (0, 0)),
            pl.BlockSpec((ku, en0), lambda i: (0, 0)),
            pl.BlockSpec((1, n_out), lambda i: (0, 0)),
        ],
        out_specs=(
            pl.BlockSpec((tb, en0), lambda i: (i, 0)),
            pl.BlockSpec((tb, en0), lambda i: (i, 0)),
        ),
        compiler_params=pltpu.CompilerParams(
            dimension_semantics=("parallel",)),
        cost_estimate=cost,
    )(x, u_flat, wx, wu, bias)
    return xh, xn


# speedup vs baseline: 1.0337x; 1.0337x over previous
import jax
import jax.numpy as jnp
from jax.experimental import pallas as pl
from jax.experimental.pallas import tpu as pltpu

_TILE_B = 2048


def _fused_two_out_kernel(x_ref, u_ref, wx_ref, wu_ref, b_ref, xh_ref, xn_ref):
    acc = jnp.dot(x_ref[...], wx_ref[...], preferred_element_type=jnp.float32)
    acc_u = jnp.dot(u_ref[...], wu_ref[...], preferred_element_type=jnp.float32)
    en0 = xh_ref.shape[1]
    xh_ref[...] = acc[:, :en0] + b_ref[:, :en0]
    xn_ref[...] = acc[:, en0:] + acc_u + b_ref[:, en0:]


def kernel(x, u_stack, w_full):
    T, batch, b0 = u_stack.shape
    en0 = x.shape[1]
    ku = T * b0
    n_out = 2 * en0

    u_flat = jnp.transpose(u_stack, (1, 0, 2)).reshape(batch, ku)
    u_flat = u_flat.astype(jnp.bfloat16)

    wx = w_full[:en0, :n_out]
    wu = w_full[en0:en0 + ku, en0:n_out].astype(jnp.bfloat16)
    bias = w_full[en0 + ku:en0 + ku + 1, :n_out]

    tb = _TILE_B if batch % _TILE_B == 0 else batch
    grid = (batch // tb,)

    cost = pl.CostEstimate(
        flops=2 * batch * (en0 * n_out + ku * en0),
        transcendentals=0,
        bytes_accessed=4 * (batch * (en0 + ku + n_out)
                            + en0 * n_out + ku * en0),
    )

    xh, xn = pl.pallas_call(
        _fused_two_out_kernel,
        out_shape=(
            jax.ShapeDtypeStruct((batch, en0), x.dtype),
            jax.ShapeDtypeStruct((batch, en0), x.dtype),
        ),
        grid=grid,
        in_specs=[
            pl.BlockSpec((tb, en0), lambda i: (i, 0)),
            pl.BlockSpec((tb, ku), lambda i: (i, 0)),
            pl.BlockSpec((en0, n_out), lambda i: (0, 0)),
            pl.BlockSpec((ku, en0), lambda i: (0, 0)),
            pl.BlockSpec((1, n_out), lambda i: (0, 0)),
        ],
        out_specs=(
            pl.BlockSpec((tb, en0), lambda i: (i, 0)),
            pl.BlockSpec((tb, en0), lambda i: (i, 0)),
        ),
        compiler_params=pltpu.CompilerParams(
            dimension_semantics=("parallel",)),
        cost_estimate=cost,
    )(x, u_flat, wx, wu, bias)
    return xh, xn


# speedup vs baseline: 1.2767x; 1.2350x over previous
import jax
import jax.numpy as jnp
from jax.experimental import pallas as pl
from jax.experimental.pallas import tpu as pltpu

_TILE_B = 2048


def _fused_two_out_kernel(x_ref, u_ref, wx_ref, wu_ref, b_ref, xh_ref, xn_ref):
    acc = jnp.dot(x_ref[...], wx_ref[...], preferred_element_type=jnp.float32)
    acc_u = jnp.dot(u_ref[...], wu_ref[...].astype(jnp.bfloat16),
                    preferred_element_type=jnp.float32)
    en0 = xh_ref.shape[1]
    b = b_ref[0:1, :]
    xh_ref[...] = acc[:, :en0] + b[:, :en0]
    xn_ref[...] = acc[:, en0:] + acc_u + b[:, en0:]


def kernel(x, u_stack, w_full):
    T, batch, b0 = u_stack.shape
    en0 = x.shape[1]
    ku = T * b0
    n_out = 2 * en0

    u_flat = jnp.transpose(u_stack, (1, 0, 2)).reshape(batch, ku)
    u_flat = u_flat.astype(jnp.bfloat16)

    tb = _TILE_B if batch % _TILE_B == 0 else batch
    grid = (batch // tb,)

    cost = pl.CostEstimate(
        flops=2 * batch * (en0 * n_out + ku * en0),
        transcendentals=0,
        bytes_accessed=4 * (batch * (en0 + n_out) + en0 * n_out + ku * en0)
        + 2 * batch * ku,
    )

    wu_row_blk = en0 // ku
    bias_blk = (en0 + ku) // 8

    xh, xn = pl.pallas_call(
        _fused_two_out_kernel,
        out_shape=(
            jax.ShapeDtypeStruct((batch, en0), x.dtype),
            jax.ShapeDtypeStruct((batch, en0), x.dtype),
        ),
        grid=grid,
        in_specs=[
            pl.BlockSpec((tb, en0), lambda i: (i, 0)),
            pl.BlockSpec((tb, ku), lambda i: (i, 0)),
            pl.BlockSpec((en0, n_out), lambda i: (0, 0)),
            pl.BlockSpec((ku, en0), lambda i: (wu_row_blk, 1)),
            pl.BlockSpec((8, n_out), lambda i: (bias_blk, 0)),
        ],
        out_specs=(
            pl.BlockSpec((tb, en0), lambda i: (i, 0)),
            pl.BlockSpec((tb, en0), lambda i: (i, 0)),
        ),
        compiler_params=pltpu.CompilerParams(
            dimension_semantics=("parallel",)),
        cost_estimate=cost,
    )(x, u_flat, w_full, w_full, w_full)
    return xh, xn
